# baseline (speedup 1.0000x reference)
import functools

import jax
import jax.numpy as jnp
from jax.experimental import pallas as pl
from jax.experimental.pallas import tpu as pltpu


def _pick_bm(n: int) -> int:
    best = 16
    for bm in range(16, 513, 16):
        if n % bm == 0 and (bm // 2) % 8 == 0:
            best = bm
    return best


def _gcn_body(x_ref, adj_a_ref, adj_b_ref, w1_ref, b1_ref, w2_ref, b2_ref,
              out_ref, s1_ref, s2_ref, h_ref, *, bm: int):
    p = pl.program_id(0)
    m = pl.program_id(1)
    hb = bm // 2

    @pl.when((p == 0) & (m == 0))
    def _():
        s1 = jnp.dot(x_ref[...].astype(jnp.bfloat16),
                     w1_ref[...].astype(jnp.bfloat16),
                     preferred_element_type=jnp.float32)
        s1_ref[...] = s1.astype(jnp.bfloat16)

    adj_a = adj_a_ref[...].astype(jnp.bfloat16)
    adj_b = adj_b_ref[...].astype(jnp.bfloat16)

    @pl.when(p == 0)
    def _():
        s1v = s1_ref[...]
        acc_a = jnp.dot(adj_a, s1v, preferred_element_type=jnp.float32)
        acc_b = jnp.dot(adj_b, s1v, preferred_element_type=jnp.float32)
        h_a = jnp.maximum(acc_a + b1_ref[...], 0.0)
        h_b = jnp.maximum(acc_b + b1_ref[...], 0.0)
        h_ref[pl.ds(m * bm, hb), :] = h_a.astype(jnp.bfloat16)
        h_ref[pl.ds(m * bm + hb, hb), :] = h_b.astype(jnp.bfloat16)

    @pl.when((p == 1) & (m == 0))
    def _():
        s2 = jnp.dot(h_ref[...], w2_ref[...].astype(jnp.bfloat16),
                     preferred_element_type=jnp.float32)
        s2_ref[...] = s2.astype(jnp.bfloat16)

    @pl.when(p == 1)
    def _():
        s2v = s2_ref[...]
        acc_a = jnp.dot(adj_a, s2v, preferred_element_type=jnp.float32)
        acc_b = jnp.dot(adj_b, s2v, preferred_element_type=jnp.float32)
        out_ref[pl.ds(0, hb), :] = acc_a + b2_ref[...]
        out_ref[pl.ds(hb, hb), :] = acc_b + b2_ref[...]


@jax.jit
def kernel(x, adj, W1, b1, W2, b2):
    n, nfeat = x.shape
    nhid = W1.shape[1]
    nout = W2.shape[1]
    bm = _pick_bm(n)
    hb = bm // 2
    grid = (2, n // bm)

    b1r = b1.reshape(1, nhid)
    b2r = b2.reshape(1, nout)

    return pl.pallas_call(
        functools.partial(_gcn_body, bm=bm),
        grid=grid,
        in_specs=[
            pl.BlockSpec((n, nfeat), lambda p, m: (0, 0)),
            pl.BlockSpec((hb, n), lambda p, m: (2 * m, 0)),
            pl.BlockSpec((hb, n), lambda p, m: (2 * m + 1, 0)),
            pl.BlockSpec((nfeat, nhid), lambda p, m: (0, 0)),
            pl.BlockSpec((1, nhid), lambda p, m: (0, 0)),
            pl.BlockSpec((nhid, nout), lambda p, m: (0, 0)),
            pl.BlockSpec((1, nout), lambda p, m: (0, 0)),
        ],
        out_specs=pl.BlockSpec((bm, nout), lambda p, m: (m, 0)),
        out_shape=jax.ShapeDtypeStruct((n, nout), jnp.float32),
        scratch_shapes=[
            pltpu.VMEM((n, nhid), jnp.bfloat16),
            pltpu.VMEM((n, nout), jnp.bfloat16),
            pltpu.VMEM((n, nhid), jnp.bfloat16),
        ],
        compiler_params=pltpu.CompilerParams(
            dimension_semantics=("arbitrary", "arbitrary"),
        ),
    )(x, adj, adj, W1, b1r, W2, b2r)
